# baseline (speedup 1.0000x reference)
import numpy as np
import jax
import jax.numpy as jnp
from jax.experimental import pallas as pl

_H_OUT = 13
_W_OUT = 13
_H_IN = 7

_BASE = np.array([[1,0],[3,0],[5,0],[7,0],[9,0],[11,0],[0,2],[2,2],[4,2],[6,2],[8,2],[10,2],[12,2],[1,4],[3,4],[5,4],[7,4],[9,4],[11,4],[2,6],[4,6],[6,6],[8,6],[10,6],[3,8],[5,8],[7,8],[9,8],[4,10],[6,10],[8,10],[5,12],[7,12]], dtype=np.int64)
_m = _BASE // 2
_MAPTO = np.stack([_m[:, 0] + (_m[:, 1] + 1) % 2, _m[:, 1]], axis=1)

_EVEN = np.array([[4,0],[6,0],[10,0],[2,0],[8,0],[5,2],[7,2],[3,2],[9,2],[1,2],[11,2],[2,4],[8,4],[10,4],[6,4],[4,4],[7,6],[9,6],[5,6],[3,6],[4,8],[6,8],[8,8],[5,10],[7,10],[6,12]], dtype=np.int64)
_EVEN_HALF = _EVEN // 2

_UNEVEN = np.array([[5,1],[6,1],[7,1],[3,1],[0,1],[4,1],[9,1],[2,1],[10,1],[1,1],[11,1],[8,1],[6,3],[3,3],[7,3],[4,3],[8,3],[2,3],[9,3],[1,3],[10,3],[0,3],[11,3],[5,3],[6,5],[4,5],[10,5],[1,5],[9,5],[5,5],[2,5],[8,5],[7,5],[3,5],[4,7],[6,7],[9,7],[5,7],[8,7],[3,7],[7,7],[2,7],[6,9],[5,9],[7,9],[8,9],[3,9],[4,9],[4,11],[7,11],[5,11],[6,11]], dtype=np.int64)
_UNEVEN_AVG = np.array([[[ii, max(jj - 1, 0)], [ii, min(jj + 1, _W_OUT - 1)], [min(ii + 1, _H_OUT - 1), max(jj - 1, 0)], [min(ii + 1, _H_OUT - 1), min(jj + 1, _W_OUT - 1)]] for ii, jj in _UNEVEN], dtype=np.int64)

_EVEN_R0 = np.minimum(_EVEN_HALF[:, 0], _H_IN - 1)
_EVEN_R1 = np.minimum(_EVEN_HALF[:, 0] + 1, _H_IN - 1)


def _build_linmap() -> np.ndarray:
    M = np.zeros((169, 49), np.float64)
    written = np.zeros((_H_OUT, _W_OUT), bool)
    for k in range(len(_BASE)):
        r, c = _BASE[k]
        mr, mc = _MAPTO[k]
        M[r * 13 + c, mr * 7 + mc] = 1.0
        written[r, c] = True
    rows = []
    for k in range(len(_UNEVEN)):
        coeffs = np.zeros(49)
        cnt = 0
        for s in range(4):
            nr, nc = _UNEVEN_AVG[k, s]
            coeffs = coeffs + M[nr * 13 + nc]
            if written[nr, nc]:
                cnt += 1
        rows.append(coeffs / max(cnt, 1))
    for k in range(len(_UNEVEN)):
        r, c = _UNEVEN[k]
        M[r * 13 + c] = rows[k]
    for k in range(len(_EVEN)):
        r, c = _EVEN[k]
        coeffs = np.zeros(49)
        coeffs[_EVEN_R0[k] * 7 + _EVEN_HALF[k, 1]] += 1
        coeffs[_EVEN_R1[k] * 7 + _EVEN_HALF[k, 1]] += 1
        M[r * 13 + c] = coeffs / 2.0
    return M


_LINMAP_T = _build_linmap().T.astype(np.float32)

_TILE = 1024


def _body(x_ref, m_ref, o_ref):
    o_ref[...] = jnp.dot(x_ref[...], m_ref[...], preferred_element_type=jnp.float32)


def kernel(input):
    B, C = input.shape[0], input.shape[1]
    rows = B * C
    x2 = input.reshape(rows, 49)
    mt = jnp.asarray(_LINMAP_T)
    out = pl.pallas_call(
        _body,
        grid=(rows // _TILE,),
        in_specs=[
            pl.BlockSpec((_TILE, 49), lambda i: (i, 0)),
            pl.BlockSpec((49, 169), lambda i: (0, 0)),
        ],
        out_specs=pl.BlockSpec((_TILE, 169), lambda i: (i, 0)),
        out_shape=jax.ShapeDtypeStruct((rows, 169), jnp.float32),
    )(x2, mt)
    return out.reshape(B, C, _H_OUT, _W_OUT)
